# baseline (speedup 1.0000x reference)
import jax
import jax.numpy as jnp
from jax.experimental import pallas as pl
from jax.experimental.pallas import tpu as pltpu
from jax.experimental.pallas import tpu_sc as plsc

_WINDOW = 256
_STREAMS = 8


def kernel(x, table):
    B, S = x.shape
    V, D = table.shape
    N = B * S
    idx = x.T.reshape(1, N)
    mesh = plsc.VectorSubcoreMesh(core_axis_name="core", subcore_axis_name="subcore")
    sub = _WINDOW // _STREAMS

    @pl.kernel(
        out_type=jax.ShapeDtypeStruct((N, D), table.dtype),
        mesh=mesh,
        scratch_types=[pltpu.SemaphoreType.DMA((_STREAMS,))],
    )
    def gather_kernel(table_hbm, i_hbm, o_hbm, sems):
        def body(i_vmem, o_vmem):
            copies = [
                pltpu.async_copy(
                    table_hbm.at[i_vmem.at[0, pl.ds(k * sub, sub)]],
                    o_vmem.at[pl.ds(k * sub, sub)],
                    sems.at[k],
                )
                for k in range(_STREAMS)
            ]
            for c in copies:
                c.wait()

        pltpu.emit_pipeline(
            body,
            grid=(N // _WINDOW,),
            in_specs=[pl.BlockSpec((1, _WINDOW), index_map=lambda i: (0, i))],
            out_specs=[pl.BlockSpec((_WINDOW, D), index_map=lambda i: (i, 0))],
            core_axis_name=("core", "subcore"),
            dimension_semantics=(pltpu.PARALLEL,),
        )(i_hbm, o_hbm)

    out2d = gather_kernel(table, idx)
    return out2d.reshape(S, B, D).transpose(1, 0, 2)


# speedup vs baseline: 1.0178x; 1.0178x over previous
import jax
import jax.numpy as jnp
from jax.experimental import pallas as pl
from jax.experimental.pallas import tpu as pltpu
from jax.experimental.pallas import tpu_sc as plsc

_WINDOW = 256
_STREAMS = 2


def kernel(x, table):
    B, S = x.shape
    V, D = table.shape
    N = B * S
    idx = x.T.reshape(1, N)
    mesh = plsc.VectorSubcoreMesh(core_axis_name="core", subcore_axis_name="subcore")
    sub = _WINDOW // _STREAMS

    @pl.kernel(
        out_type=jax.ShapeDtypeStruct((N, D), table.dtype),
        mesh=mesh,
        scratch_types=[pltpu.SemaphoreType.DMA((_STREAMS,))],
    )
    def gather_kernel(table_hbm, i_hbm, o_hbm, sems):
        def body(i_vmem, o_vmem):
            copies = [
                pltpu.async_copy(
                    table_hbm.at[i_vmem.at[0, pl.ds(k * sub, sub)]],
                    o_vmem.at[pl.ds(k * sub, sub)],
                    sems.at[k],
                )
                for k in range(_STREAMS)
            ]
            for c in copies:
                c.wait()

        pltpu.emit_pipeline(
            body,
            grid=(N // _WINDOW,),
            in_specs=[pl.BlockSpec((1, _WINDOW), index_map=lambda i: (0, i))],
            out_specs=[pl.BlockSpec((_WINDOW, D), index_map=lambda i: (i, 0))],
            core_axis_name=("core", "subcore"),
            dimension_semantics=(pltpu.PARALLEL,),
        )(i_hbm, o_hbm)

    out2d = gather_kernel(table, idx)
    return out2d.reshape(S, B, D).transpose(1, 0, 2)
